# baseline (speedup 1.0000x reference)
import functools

import jax
import jax.numpy as jnp
from jax import lax
from jax.experimental import pallas as pl
from jax.experimental.pallas import tpu as pltpu
from jax.experimental.pallas import tpu_sc as plsc

N = 2048
E = 8192
M = 512
TAU = 0.07
INV_TAU = 1.0 / TAU
NEG = -1e30
NTILES = 16
NPT = N // NTILES
EPT = M // NTILES
LN2 = 0.6931471805599453
EB = 2048
NB = E // EB
CCAP = 2 * (E + 4 * NPT)


def _log16(x):
    bits = plsc.bitcast(x, jnp.int32)
    e = ((bits >> 23) & 0xFF) - 127
    m = plsc.bitcast((bits & 0x7FFFFF) | jnp.int32(0x3F800000), jnp.float32)
    big = m > 1.4142135381698608
    m = jnp.where(big, m * 0.5, m)
    e = jnp.where(big, e + 1, e)
    u = (m - 1.0) / (m + 1.0)
    u2 = u * u
    p = u * (2.0 + u2 * (0.66666666666 + u2 * (0.4 + u2 * 0.28571428571)))
    return e.astype(jnp.float32) * LN2 + p


_MESH = plsc.VectorSubcoreMesh(core_axis_name="c", subcore_axis_name="s",
                               num_cores=1)


@functools.partial(
    pl.kernel,
    out_type=(
        jax.ShapeDtypeStruct((2 * N,), jnp.float32),
        jax.ShapeDtypeStruct((2 * M,), jnp.float32),
        jax.ShapeDtypeStruct((2 * M,), jnp.float32),
    ),
    mesh=_MESH,
    compiler_params=pltpu.CompilerParams(needs_layout_passes=False),
    scratch_types=[
        pltpu.VMEM((E,), jnp.int32),
        pltpu.VMEM((E,), jnp.int32),
        pltpu.VMEM((E,), jnp.int32),
        pltpu.VMEM((CCAP,), jnp.int32),
        pltpu.VMEM((CCAP,), jnp.float32),
        pltpu.VMEM((CCAP,), jnp.float32),
        pltpu.VMEM((4 * N,), jnp.float32),
        pltpu.VMEM((2 * N,), jnp.float32),
        pltpu.VMEM((NPT,), jnp.int32),
        pltpu.VMEM((NPT,), jnp.int32),
        pltpu.VMEM((16 * NPT,), jnp.int32),
        pltpu.VMEM((EB,), jnp.int32),
        pltpu.VMEM((EB,), jnp.int32),
        pltpu.VMEM((8 * EB,), jnp.float32),
        pltpu.VMEM((EPT,), jnp.int32),
        pltpu.VMEM((2 * EPT,), jnp.float32),
        pltpu.VMEM((2 * EPT,), jnp.float32),
        pltpu.VMEM((2 * EPT,), jnp.float32),
        pltpu.VMEM_SHARED((4 * N,), jnp.float32),
        pltpu.VMEM_SHARED((2 * N,), jnp.float32),
    ],
)
def _sta_sc(dh, ms, esrc, edst, inarr, epids, rat,
            out_at, out_ep, out_slack,
            packed, leids, posv, cidx, cwr, cwf, mssl, atf,
            cnts, offs, h, sc_a, sc_b, dmcols,
            epv, ratv, atepv, slkv, mssh, atsh):
    sid = lax.axis_index("s")
    lo = sid * NPT
    hi = lo + NPT
    it16 = lax.iota(jnp.int32, 16)
    zero16i = jnp.zeros((16,), jnp.int32)
    negv = jnp.full((16,), NEG, jnp.float32)
    zf16 = jnp.zeros((16,), jnp.float32)
    lane0 = it16 == 0

    def p1(b, nloc):
        pltpu.sync_copy(esrc.at[pl.ds(b * EB, EB)], sc_a)
        pltpu.sync_copy(edst.at[pl.ds(b * EB, EB)], sc_b)

        def inner(c, nl):
            s = sc_a[pl.ds(c * 16, 16)]
            d = sc_b[pl.ds(c * 16, 16)]
            packed[pl.ds(b * EB + c * 16, 16)] = s | ((d - s) << 16)
            mk = (d >= lo) & (d < hi)
            mi = mk.astype(jnp.int32)
            cs = plsc.cumsum(mi)
            val = (b * EB + c * 16 + it16) | ((d - lo) << 16)
            plsc.store_scatter(leids, [nl + cs - 1], val, mask=mk)
            return nl + jnp.sum(mi)

        return lax.fori_loop(0, EB // 16, inner, nloc)

    nloc = lax.fori_loop(0, NB, p1, jnp.int32(0))
    ntk = (nloc + 15) >> 4

    def zh(k, _):
        for q in range(16):
            h[pl.ds(k * 256 + q * 16, 16)] = zero16i
        return 0

    lax.fori_loop(0, NPT // 16, zh, 0)

    def pA(t, _):
        lv = leids[pl.ds(t * 16, 16)]
        msk = (t * 16 + it16) < nloc
        v = lv >> 16
        addr = it16 * NPT + v
        c = plsc.load_gather(h, [addr], mask=msk)
        plsc.store_scatter(h, [addr], c + 1, mask=msk)
        return 0

    lax.fori_loop(0, ntk, pA, 0)

    def red(kb, _):
        def rl(L, acc):
            return acc + h[pl.ds(L * NPT + kb * 16, 16)]

        cnts[pl.ds(kb * 16, 16)] = lax.fori_loop(0, 16, rl, zero16i)
        return 0

    lax.fori_loop(0, NPT // 16, red, 0)

    carry = jnp.int32(0)
    for kb in range(NPT // 16):
        v = cnts[pl.ds(kb * 16, 16)]
        a = (v + 3) & ~jnp.int32(3)
        cs = plsc.cumsum(a)
        offs[pl.ds(kb * 16, 16)] = cs - a + carry
        carry = carry + jnp.max(cs)

    def rn(kb, _):
        def rl(L, run):
            tmp = h[pl.ds(L * NPT + kb * 16, 16)]
            h[pl.ds(L * NPT + kb * 16, 16)] = run
            return run + tmp

        lax.fori_loop(0, 16, rl, offs[pl.ds(kb * 16, 16)])
        return 0

    lax.fori_loop(0, NPT // 16, rn, 0)

    def pB(t, _):
        lv = leids[pl.ds(t * 16, 16)]
        msk = (t * 16 + it16) < nloc
        ev = lv & 0xFFFF
        v = lv >> 16
        addr = it16 * NPT + v
        p = plsc.load_gather(h, [addr], mask=msk)
        plsc.store_scatter(h, [addr], p + 1, mask=msk)
        plsc.store_scatter(posv, [ev], p, mask=msk)
        return 0

    lax.fori_loop(0, ntk, pB, 0)

    def p5(b, _):
        for c in range(4):
            pltpu.sync_copy(dh.at[pl.ds(c * E + b * EB, EB)],
                            dmcols.at[pl.ds(c * EB, EB)])
            pltpu.sync_copy(ms.at[pl.ds(c * E + b * EB, EB)],
                            dmcols.at[pl.ds((4 + c) * EB, EB)])

        def inner(c, _):
            pk = packed[pl.ds(b * EB + c * 16, 16)]
            s = pk & 0xFFFF
            dv = s + (pk >> 16)
            mk = (dv >= lo) & (dv < hi)
            p = posv[pl.ds(b * EB + c * 16, 16)]
            d0 = dmcols[pl.ds(c * 16, 16)]
            d1 = dmcols[pl.ds(EB + c * 16, 16)]
            d2 = dmcols[pl.ds(2 * EB + c * 16, 16)]
            d3 = dmcols[pl.ds(3 * EB + c * 16, 16)]
            m0 = dmcols[pl.ds(4 * EB + c * 16, 16)]
            m1 = dmcols[pl.ds(5 * EB + c * 16, 16)]
            m2 = dmcols[pl.ds(6 * EB + c * 16, 16)]
            m3 = dmcols[pl.ds(7 * EB + c * 16, 16)]
            s0 = 2 * p
            s1 = s0 + 1
            gi = 4 * s
            plsc.store_scatter(cidx, [s0], gi, mask=mk)
            plsc.store_scatter(cidx, [s1], gi + 2, mask=mk)
            plsc.store_scatter(cwr, [s0], jnp.where(m0 > 0.5, d0, negv),
                               mask=mk)
            plsc.store_scatter(cwr, [s1], jnp.where(m2 > 0.5, d2, negv),
                               mask=mk)
            plsc.store_scatter(cwf, [s0], jnp.where(m1 > 0.5, d1, negv),
                               mask=mk)
            plsc.store_scatter(cwf, [s1], jnp.where(m3 > 0.5, d3, negv),
                               mask=mk)
            return 0

        return lax.fori_loop(0, EB // 16, inner, 0)

    lax.fori_loop(0, NB, p5, 0)

    ones16 = jnp.ones((16,), jnp.float32)
    pltpu.sync_copy(inarr.at[pl.ds(2 * lo, 2 * NPT)],
                    atf.at[pl.ds(2 * lo, 2 * NPT)])
    for k in range(2 * NPT // 16):
        jl = 2 * lo + k * 16 + it16
        plsc.store_scatter(mssl, [2 * jl], atf[pl.ds(2 * lo + k * 16, 16)])
        plsc.store_scatter(mssl, [2 * jl + 1], ones16)

    @pl.when(sid > 0)
    def _init_halo():
        pltpu.sync_copy(inarr.at[pl.ds(2 * lo - 128, 128)],
                        atf.at[pl.ds(2 * lo - 128, 128)])
        for k in range(128 // 16):
            jl = 2 * lo - 128 + k * 16 + it16
            plsc.store_scatter(mssl, [2 * jl],
                               atf[pl.ds(2 * lo - 128 + k * 16, 16)])
            plsc.store_scatter(mssl, [2 * jl + 1], ones16)

    lt2 = it16 < 2
    lt4 = it16 < 4

    def stage_body(stage, _):
        @pl.when(stage == sid)
        def _active():
            @pl.when(stage > 0)
            def _pull():
                pltpu.sync_copy(mssh.at[pl.ds(4 * lo - 256, 256)],
                                mssl.at[pl.ds(4 * lo - 256, 256)])

            def one_node(i, cnt, o8):
                cnt2 = 2 * cnt
                valid = it16 < cnt2
                gi = cidx[pl.ds(o8 * 8, 16)]
                wr = cwr[pl.ds(o8 * 8, 16)]
                wf = cwf[pl.ds(o8 * 8, 16)]
                mu = plsc.load_gather(mssl, [gi], mask=valid)
                su = plsc.load_gather(mssl, [gi + 1], mask=valid)
                rmr = jnp.where(valid, mu + wr, negv)
                rmf = jnp.where(valid, mu + wf, negv)
                rsr = jnp.where(valid, su, zf16)
                rsf = rsr

                def chunk(cc, carr):
                    crmr, crsr, crmf, crsf = carr
                    cvalid = (cc * 16 + it16) < cnt2
                    cgi = cidx[pl.ds(o8 * 8 + cc * 16, 16)]
                    cwrv = cwr[pl.ds(o8 * 8 + cc * 16, 16)]
                    cwfv = cwf[pl.ds(o8 * 8 + cc * 16, 16)]
                    cmu = plsc.load_gather(mssl, [cgi], mask=cvalid)
                    csu = jnp.where(
                        cvalid,
                        plsc.load_gather(mssl, [cgi + 1], mask=cvalid),
                        zf16)
                    vr = jnp.where(cvalid, cmu + cwrv, negv)
                    vf = jnp.where(cvalid, cmu + cwfv, negv)
                    nmr = jnp.maximum(crmr, vr)
                    nsr = crsr * jnp.exp((crmr - nmr) * INV_TAU) + \
                        csu * jnp.exp((vr - nmr) * INV_TAU)
                    nmf = jnp.maximum(crmf, vf)
                    nsf = crsf * jnp.exp((crmf - nmf) * INV_TAU) + \
                        csu * jnp.exp((vf - nmf) * INV_TAU)
                    return (nmr, nsr, nmf, nsf)

                rmr, rsr, rmf, rsf = lax.fori_loop(
                    1, (cnt2 + 15) >> 4, chunk, (rmr, rsr, rmf, rsf))

                mr = jnp.max(rmr)
                mrv = jnp.full((16,), mr, jnp.float32)
                srv = jnp.full((16,), jnp.sum(
                    rsr * jnp.exp((rmr - mrv) * INV_TAU)), jnp.float32)
                mf = jnp.max(rmf)
                mfv = jnp.full((16,), mf, jnp.float32)
                sfv = jnp.full((16,), jnp.sum(
                    rsf * jnp.exp((rmf - mfv) * INV_TAU)), jnp.float32)
                sv = jnp.where(lt2, srv, sfv)
                bits = plsc.bitcast(sv, jnp.int32)
                kk = ((bits >> 23) & 0xFF) - 127
                tt = plsc.bitcast((bits & 0x7FFFFF) | jnp.int32(0x3F800000),
                                  jnp.float32)
                mm = jnp.where(lt2, mrv, mfv) + \
                    (TAU * LN2) * kk.astype(jnp.float32)
                vals = jnp.where((it16 & 1) == 0, mm, tt)
                ok = lt4 & jnp.where(lt2, mr > -1e20, mf > -1e20)
                plsc.store_scatter(mssl, [4 * (lo + i) + it16], vals,
                                   mask=ok)
                return o8 + ((cnt + 3) >> 2)

            def node_body(kb, o8):
                cv = cnts[pl.ds(kb * 16, 16)]
                for k in range(16):
                    o8 = one_node(kb * 16 + k, cv[k], o8)
                return o8

            lax.fori_loop(0, NPT // 16, node_body, jnp.int32(0))
            pltpu.sync_copy(mssl.at[pl.ds(4 * lo + 4 * NPT - 256, 256)],
                            mssh.at[pl.ds(4 * lo + 4 * NPT - 256, 256)])

        plsc.subcore_barrier()
        return 0

    lax.fori_loop(0, NTILES, stage_body, 0)

    for c in range(2 * NPT // 16):
        jl = 2 * lo + c * 16 + it16
        mval = plsc.load_gather(mssl, [2 * jl])
        sval = plsc.load_gather(mssl, [2 * jl + 1])
        atf[pl.ds(2 * lo + c * 16, 16)] = mval + TAU * _log16(sval)
    pltpu.sync_copy(atf.at[pl.ds(2 * lo, 2 * NPT)],
                    atsh.at[pl.ds(2 * lo, 2 * NPT)])
    pltpu.sync_copy(atf.at[pl.ds(2 * lo, 2 * NPT)],
                    out_at.at[pl.ds(2 * lo, 2 * NPT)])
    plsc.subcore_barrier()

    pltpu.sync_copy(atsh, atf)
    pltpu.sync_copy(epids.at[pl.ds(sid * EPT, EPT)], epv)
    pltpu.sync_copy(rat.at[pl.ds(sid * 2 * EPT, 2 * EPT)], ratv)
    for c in range(2 * EPT // 16):
        jl = c * 16 + it16
        row = jl >> 1
        ch = jl & 1
        ep = plsc.load_gather(epv, [row])
        a = plsc.load_gather(atf, [2 * ep + ch])
        r = plsc.load_gather(ratv, [jl])
        plsc.store_scatter(atepv, [jl], a)
        plsc.store_scatter(slkv, [jl], r - a)
    pltpu.sync_copy(atepv, out_ep.at[pl.ds(sid * 2 * EPT, 2 * EPT)])
    pltpu.sync_copy(slkv, out_slack.at[pl.ds(sid * 2 * EPT, 2 * EPT)])


def kernel(d_hat, sta_mask, edge_src, edge_dst, topo_order, input_arrival,
           endpoint_ids, rat_true):
    del topo_order
    at_flat, ep_flat, slk_flat = _sta_sc(
        d_hat.T.reshape(-1), sta_mask.T.reshape(-1), edge_src, edge_dst,
        input_arrival.reshape(-1), endpoint_ids, rat_true.reshape(-1))
    return (at_flat.reshape(N, 2), ep_flat.reshape(M, 2),
            slk_flat.reshape(M, 2))


# speedup vs baseline: 1.1576x; 1.1576x over previous
import functools

import jax
import jax.numpy as jnp
from jax import lax
from jax.experimental import pallas as pl
from jax.experimental.pallas import tpu as pltpu
from jax.experimental.pallas import tpu_sc as plsc

N = 2048
E = 8192
M = 512
TAU = 0.07
INV_TAU = 1.0 / TAU
NEG = -1e30
NTILES = 16
NPT = N // NTILES
EPT = M // NTILES
LN2 = 0.6931471805599453
EB = 2048
NB = E // EB
CCAP = 2 * (E + 4 * NPT)


def _log16(x):
    bits = plsc.bitcast(x, jnp.int32)
    e = ((bits >> 23) & 0xFF) - 127
    m = plsc.bitcast((bits & 0x7FFFFF) | jnp.int32(0x3F800000), jnp.float32)
    big = m > 1.4142135381698608
    m = jnp.where(big, m * 0.5, m)
    e = jnp.where(big, e + 1, e)
    u = (m - 1.0) / (m + 1.0)
    u2 = u * u
    p = u * (2.0 + u2 * (0.66666666666 + u2 * (0.4 + u2 * 0.28571428571)))
    return e.astype(jnp.float32) * LN2 + p


_MESH = plsc.VectorSubcoreMesh(core_axis_name="c", subcore_axis_name="s",
                               num_cores=1)


@functools.partial(
    pl.kernel,
    out_type=(
        jax.ShapeDtypeStruct((2 * N,), jnp.float32),
        jax.ShapeDtypeStruct((2 * M,), jnp.float32),
        jax.ShapeDtypeStruct((2 * M,), jnp.float32),
    ),
    mesh=_MESH,
    compiler_params=pltpu.CompilerParams(needs_layout_passes=False),
    scratch_types=[
        pltpu.VMEM((E,), jnp.int32),
        pltpu.VMEM((E,), jnp.int32),
        pltpu.VMEM((E,), jnp.int32),
        pltpu.VMEM((CCAP,), jnp.int32),
        pltpu.VMEM((CCAP,), jnp.float32),
        pltpu.VMEM((CCAP,), jnp.float32),
        pltpu.VMEM((4 * N,), jnp.float32),
        pltpu.VMEM((2 * N,), jnp.float32),
        pltpu.VMEM((NPT,), jnp.int32),
        pltpu.VMEM((NPT,), jnp.int32),
        pltpu.VMEM((16 * NPT,), jnp.int32),
        pltpu.VMEM((EB,), jnp.int32),
        pltpu.VMEM((EB,), jnp.int32),
        pltpu.VMEM((8 * EB,), jnp.float32),
        pltpu.VMEM((EPT,), jnp.int32),
        pltpu.VMEM((2 * EPT,), jnp.float32),
        pltpu.VMEM((2 * EPT,), jnp.float32),
        pltpu.VMEM((2 * EPT,), jnp.float32),
        pltpu.VMEM_SHARED((4 * N,), jnp.float32),
        pltpu.VMEM_SHARED((2 * N,), jnp.float32),
    ],
)
def _sta_sc(dh, ms, esrc, edst, inarr, epids, rat,
            out_at, out_ep, out_slack,
            packed, leids, posv, cidx, cwr, cwf, mssl, atf,
            cnts, offs, h, sc_a, sc_b, dmcols,
            epv, ratv, atepv, slkv, mssh, atsh):
    sid = lax.axis_index("s")
    lo = sid * NPT
    hi = lo + NPT
    it16 = lax.iota(jnp.int32, 16)
    zero16i = jnp.zeros((16,), jnp.int32)
    negv = jnp.full((16,), NEG, jnp.float32)
    zf16 = jnp.zeros((16,), jnp.float32)
    lane0 = it16 == 0

    def p1(b, nloc):
        pltpu.sync_copy(esrc.at[pl.ds(b * EB, EB)], sc_a)
        pltpu.sync_copy(edst.at[pl.ds(b * EB, EB)], sc_b)

        def inner(c, nl):
            s = sc_a[pl.ds(c * 16, 16)]
            d = sc_b[pl.ds(c * 16, 16)]
            packed[pl.ds(b * EB + c * 16, 16)] = s | ((d - s) << 16)
            mk = (d >= lo) & (d < hi)
            mi = mk.astype(jnp.int32)
            cs = plsc.cumsum(mi)
            val = (b * EB + c * 16 + it16) | ((d - lo) << 16)
            plsc.store_scatter(leids, [nl + cs - 1], val, mask=mk)
            return nl + jnp.sum(mi)

        return lax.fori_loop(0, EB // 16, inner, nloc)

    nloc = lax.fori_loop(0, NB, p1, jnp.int32(0))
    ntk = (nloc + 15) >> 4

    def zh(k, _):
        for q in range(16):
            h[pl.ds(k * 256 + q * 16, 16)] = zero16i
        return 0

    lax.fori_loop(0, NPT // 16, zh, 0)

    def pA(t, _):
        lv = leids[pl.ds(t * 16, 16)]
        msk = (t * 16 + it16) < nloc
        v = lv >> 16
        addr = it16 * NPT + v
        c = plsc.load_gather(h, [addr], mask=msk)
        plsc.store_scatter(h, [addr], c + 1, mask=msk)
        return 0

    lax.fori_loop(0, ntk, pA, 0)

    def red(kb, _):
        def rl(L, acc):
            return acc + h[pl.ds(L * NPT + kb * 16, 16)]

        cnts[pl.ds(kb * 16, 16)] = lax.fori_loop(0, 16, rl, zero16i)
        return 0

    lax.fori_loop(0, NPT // 16, red, 0)

    carry = jnp.int32(0)
    for kb in range(NPT // 16):
        v = cnts[pl.ds(kb * 16, 16)]
        a = (v + 3) & ~jnp.int32(3)
        cs = plsc.cumsum(a)
        offs[pl.ds(kb * 16, 16)] = cs - a + carry
        carry = carry + jnp.max(cs)

    def rn(kb, _):
        def rl(L, run):
            tmp = h[pl.ds(L * NPT + kb * 16, 16)]
            h[pl.ds(L * NPT + kb * 16, 16)] = run
            return run + tmp

        lax.fori_loop(0, 16, rl, offs[pl.ds(kb * 16, 16)])
        return 0

    lax.fori_loop(0, NPT // 16, rn, 0)

    def pB(t, _):
        lv = leids[pl.ds(t * 16, 16)]
        msk = (t * 16 + it16) < nloc
        ev = lv & 0xFFFF
        v = lv >> 16
        addr = it16 * NPT + v
        p = plsc.load_gather(h, [addr], mask=msk)
        plsc.store_scatter(h, [addr], p + 1, mask=msk)
        plsc.store_scatter(posv, [ev], p, mask=msk)
        return 0

    lax.fori_loop(0, ntk, pB, 0)

    def p5(b, _):
        for c in range(4):
            pltpu.sync_copy(dh.at[pl.ds(c * E + b * EB, EB)],
                            dmcols.at[pl.ds(c * EB, EB)])
            pltpu.sync_copy(ms.at[pl.ds(c * E + b * EB, EB)],
                            dmcols.at[pl.ds((4 + c) * EB, EB)])

        def inner(c, _):
            pk = packed[pl.ds(b * EB + c * 16, 16)]
            s = pk & 0xFFFF
            dv = s + (pk >> 16)
            mk = (dv >= lo) & (dv < hi)
            p = posv[pl.ds(b * EB + c * 16, 16)]
            d0 = dmcols[pl.ds(c * 16, 16)]
            d1 = dmcols[pl.ds(EB + c * 16, 16)]
            d2 = dmcols[pl.ds(2 * EB + c * 16, 16)]
            d3 = dmcols[pl.ds(3 * EB + c * 16, 16)]
            m0 = dmcols[pl.ds(4 * EB + c * 16, 16)]
            m1 = dmcols[pl.ds(5 * EB + c * 16, 16)]
            m2 = dmcols[pl.ds(6 * EB + c * 16, 16)]
            m3 = dmcols[pl.ds(7 * EB + c * 16, 16)]
            s0 = 2 * p
            s1 = s0 + 1
            gi = 4 * s
            plsc.store_scatter(cidx, [s0], gi, mask=mk)
            plsc.store_scatter(cidx, [s1], gi + 2, mask=mk)
            plsc.store_scatter(cwr, [s0], jnp.where(m0 > 0.5, d0, negv),
                               mask=mk)
            plsc.store_scatter(cwr, [s1], jnp.where(m2 > 0.5, d2, negv),
                               mask=mk)
            plsc.store_scatter(cwf, [s0], jnp.where(m1 > 0.5, d1, negv),
                               mask=mk)
            plsc.store_scatter(cwf, [s1], jnp.where(m3 > 0.5, d3, negv),
                               mask=mk)
            return 0

        return lax.fori_loop(0, EB // 16, inner, 0)

    lax.fori_loop(0, NB, p5, 0)

    ones16 = jnp.ones((16,), jnp.float32)
    pltpu.sync_copy(inarr.at[pl.ds(2 * lo, 2 * NPT)],
                    atf.at[pl.ds(2 * lo, 2 * NPT)])
    for k in range(2 * NPT // 16):
        jl = 2 * lo + k * 16 + it16
        plsc.store_scatter(mssl, [2 * jl], atf[pl.ds(2 * lo + k * 16, 16)])
        plsc.store_scatter(mssl, [2 * jl + 1], ones16)

    @pl.when(sid > 0)
    def _init_halo():
        pltpu.sync_copy(inarr.at[pl.ds(2 * lo - 128, 128)],
                        atf.at[pl.ds(2 * lo - 128, 128)])
        for k in range(128 // 16):
            jl = 2 * lo - 128 + k * 16 + it16
            plsc.store_scatter(mssl, [2 * jl],
                               atf[pl.ds(2 * lo - 128 + k * 16, 16)])
            plsc.store_scatter(mssl, [2 * jl + 1], ones16)

    lt2 = it16 < 2
    lt4 = it16 < 4

    def stage_body(stage, _):
        @pl.when(stage == sid)
        def _active():
            @pl.when(stage > 0)
            def _pull():
                pltpu.sync_copy(mssh.at[pl.ds(4 * lo - 256, 256)],
                                mssl.at[pl.ds(4 * lo - 256, 256)])

            def one_node(i, o8):
                iv = jnp.full((16,), i, jnp.int32)
                cnt = jnp.max(plsc.load_gather(cnts, [iv]))
                cnt2 = 2 * cnt
                valid = it16 < cnt2
                gi = cidx[pl.ds(o8 * 8, 16)]
                wr = cwr[pl.ds(o8 * 8, 16)]
                wf = cwf[pl.ds(o8 * 8, 16)]
                mu = plsc.load_gather(mssl, [gi], mask=valid)
                su = plsc.load_gather(mssl, [gi + 1], mask=valid)
                rmr = jnp.where(valid, mu + wr, negv)
                rmf = jnp.where(valid, mu + wf, negv)
                rsr = jnp.where(valid, su, zf16)
                rsf = rsr

                def chunk(cc, carr):
                    crmr, crsr, crmf, crsf = carr
                    cvalid = (cc * 16 + it16) < cnt2
                    cgi = cidx[pl.ds(o8 * 8 + cc * 16, 16)]
                    cwrv = cwr[pl.ds(o8 * 8 + cc * 16, 16)]
                    cwfv = cwf[pl.ds(o8 * 8 + cc * 16, 16)]
                    cmu = plsc.load_gather(mssl, [cgi], mask=cvalid)
                    csu = jnp.where(
                        cvalid,
                        plsc.load_gather(mssl, [cgi + 1], mask=cvalid),
                        zf16)
                    vr = jnp.where(cvalid, cmu + cwrv, negv)
                    vf = jnp.where(cvalid, cmu + cwfv, negv)
                    nmr = jnp.maximum(crmr, vr)
                    nsr = crsr * jnp.exp((crmr - nmr) * INV_TAU) + \
                        csu * jnp.exp((vr - nmr) * INV_TAU)
                    nmf = jnp.maximum(crmf, vf)
                    nsf = crsf * jnp.exp((crmf - nmf) * INV_TAU) + \
                        csu * jnp.exp((vf - nmf) * INV_TAU)
                    return (nmr, nsr, nmf, nsf)

                rmr, rsr, rmf, rsf = lax.fori_loop(
                    1, (cnt2 + 15) >> 4, chunk, (rmr, rsr, rmf, rsf))

                mr = jnp.max(rmr)
                mrv = jnp.full((16,), mr, jnp.float32)
                srv = jnp.full((16,), jnp.sum(
                    rsr * jnp.exp((rmr - mrv) * INV_TAU)), jnp.float32)
                mf = jnp.max(rmf)
                mfv = jnp.full((16,), mf, jnp.float32)
                sfv = jnp.full((16,), jnp.sum(
                    rsf * jnp.exp((rmf - mfv) * INV_TAU)), jnp.float32)
                sv = jnp.where(lt2, srv, sfv)
                bits = plsc.bitcast(sv, jnp.int32)
                kk = ((bits >> 23) & 0xFF) - 127
                tt = plsc.bitcast((bits & 0x7FFFFF) | jnp.int32(0x3F800000),
                                  jnp.float32)
                mm = jnp.where(lt2, mrv, mfv) + \
                    (TAU * LN2) * kk.astype(jnp.float32)
                vals = jnp.where((it16 & 1) == 0, mm, tt)
                ok = lt4 & jnp.where(lt2, mr > -1e20, mf > -1e20)
                plsc.store_scatter(mssl, [4 * (lo + i) + it16], vals,
                                   mask=ok)
                return o8 + ((cnt + 3) >> 2)

            def node_body(g, o8):
                return one_node(2 * g + 1, one_node(2 * g, o8))

            lax.fori_loop(0, NPT // 2, node_body, jnp.int32(0))
            pltpu.sync_copy(mssl.at[pl.ds(4 * lo + 4 * NPT - 256, 256)],
                            mssh.at[pl.ds(4 * lo + 4 * NPT - 256, 256)])

        plsc.subcore_barrier()
        return 0

    lax.fori_loop(0, NTILES, stage_body, 0)

    for c in range(2 * NPT // 16):
        jl = 2 * lo + c * 16 + it16
        mval = plsc.load_gather(mssl, [2 * jl])
        sval = plsc.load_gather(mssl, [2 * jl + 1])
        atf[pl.ds(2 * lo + c * 16, 16)] = mval + TAU * _log16(sval)
    pltpu.sync_copy(atf.at[pl.ds(2 * lo, 2 * NPT)],
                    atsh.at[pl.ds(2 * lo, 2 * NPT)])
    pltpu.sync_copy(atf.at[pl.ds(2 * lo, 2 * NPT)],
                    out_at.at[pl.ds(2 * lo, 2 * NPT)])
    plsc.subcore_barrier()

    pltpu.sync_copy(atsh, atf)
    pltpu.sync_copy(epids.at[pl.ds(sid * EPT, EPT)], epv)
    pltpu.sync_copy(rat.at[pl.ds(sid * 2 * EPT, 2 * EPT)], ratv)
    for c in range(2 * EPT // 16):
        jl = c * 16 + it16
        row = jl >> 1
        ch = jl & 1
        ep = plsc.load_gather(epv, [row])
        a = plsc.load_gather(atf, [2 * ep + ch])
        r = plsc.load_gather(ratv, [jl])
        plsc.store_scatter(atepv, [jl], a)
        plsc.store_scatter(slkv, [jl], r - a)
    pltpu.sync_copy(atepv, out_ep.at[pl.ds(sid * 2 * EPT, 2 * EPT)])
    pltpu.sync_copy(slkv, out_slack.at[pl.ds(sid * 2 * EPT, 2 * EPT)])


def kernel(d_hat, sta_mask, edge_src, edge_dst, topo_order, input_arrival,
           endpoint_ids, rat_true):
    del topo_order
    at_flat, ep_flat, slk_flat = _sta_sc(
        d_hat.T.reshape(-1), sta_mask.T.reshape(-1), edge_src, edge_dst,
        input_arrival.reshape(-1), endpoint_ids, rat_true.reshape(-1))
    return (at_flat.reshape(N, 2), ep_flat.reshape(M, 2),
            slk_flat.reshape(M, 2))


# speedup vs baseline: 1.1691x; 1.0099x over previous
import functools

import jax
import jax.numpy as jnp
from jax import lax
from jax.experimental import pallas as pl
from jax.experimental.pallas import tpu as pltpu
from jax.experimental.pallas import tpu_sc as plsc

N = 2048
E = 8192
M = 512
TAU = 0.07
INV_TAU = 1.0 / TAU
NEG = -1e30
NTILES = 16
NPT = N // NTILES
EPT = M // NTILES
LN2 = 0.6931471805599453
EB = 2048
NB = E // EB
CCAP = 2 * (E + 4 * NPT)


def _log16(x):
    bits = plsc.bitcast(x, jnp.int32)
    e = ((bits >> 23) & 0xFF) - 127
    m = plsc.bitcast((bits & 0x7FFFFF) | jnp.int32(0x3F800000), jnp.float32)
    big = m > 1.4142135381698608
    m = jnp.where(big, m * 0.5, m)
    e = jnp.where(big, e + 1, e)
    u = (m - 1.0) / (m + 1.0)
    u2 = u * u
    p = u * (2.0 + u2 * (0.66666666666 + u2 * (0.4 + u2 * 0.28571428571)))
    return e.astype(jnp.float32) * LN2 + p


_MESH = plsc.VectorSubcoreMesh(core_axis_name="c", subcore_axis_name="s",
                               num_cores=1)


@functools.partial(
    pl.kernel,
    out_type=(
        jax.ShapeDtypeStruct((2 * N,), jnp.float32),
        jax.ShapeDtypeStruct((2 * M,), jnp.float32),
        jax.ShapeDtypeStruct((2 * M,), jnp.float32),
    ),
    mesh=_MESH,
    compiler_params=pltpu.CompilerParams(needs_layout_passes=False),
    scratch_types=[
        pltpu.VMEM((E,), jnp.int32),
        pltpu.VMEM((E,), jnp.int32),
        pltpu.VMEM((E,), jnp.int32),
        pltpu.VMEM((CCAP,), jnp.int32),
        pltpu.VMEM((CCAP,), jnp.float32),
        pltpu.VMEM((CCAP,), jnp.float32),
        pltpu.VMEM((4 * N,), jnp.float32),
        pltpu.VMEM((2 * N,), jnp.float32),
        pltpu.VMEM((NPT,), jnp.int32),
        pltpu.VMEM((NPT,), jnp.int32),
        pltpu.VMEM((16 * NPT,), jnp.int32),
        pltpu.VMEM((EB,), jnp.int32),
        pltpu.VMEM((EB,), jnp.int32),
        pltpu.VMEM((8 * EB,), jnp.float32),
        pltpu.VMEM((EPT,), jnp.int32),
        pltpu.VMEM((2 * EPT,), jnp.float32),
        pltpu.VMEM((2 * EPT,), jnp.float32),
        pltpu.VMEM((2 * EPT,), jnp.float32),
        pltpu.VMEM_SHARED((4 * N,), jnp.float32),
        pltpu.VMEM_SHARED((2 * N,), jnp.float32),
    ],
)
def _sta_sc(dh, ms, esrc, edst, inarr, epids, rat,
            out_at, out_ep, out_slack,
            packed, leids, posv, cidx, cwr, cwf, mssl, atf,
            cnts, offs, h, sc_a, sc_b, dmcols,
            epv, ratv, atepv, slkv, mssh, atsh):
    sid = lax.axis_index("s")
    lo = sid * NPT
    hi = lo + NPT
    it16 = lax.iota(jnp.int32, 16)
    zero16i = jnp.zeros((16,), jnp.int32)
    negv = jnp.full((16,), NEG, jnp.float32)
    zf16 = jnp.zeros((16,), jnp.float32)
    lane0 = it16 == 0

    def p1(b, nloc):
        pltpu.sync_copy(esrc.at[pl.ds(b * EB, EB)], sc_a)
        pltpu.sync_copy(edst.at[pl.ds(b * EB, EB)], sc_b)

        def inner(c, nl):
            s = sc_a[pl.ds(c * 16, 16)]
            d = sc_b[pl.ds(c * 16, 16)]
            packed[pl.ds(b * EB + c * 16, 16)] = s | ((d - s) << 16)
            mk = (d >= lo) & (d < hi)
            mi = mk.astype(jnp.int32)
            cs = plsc.cumsum(mi)
            val = (b * EB + c * 16 + it16) | ((d - lo) << 16)
            plsc.store_scatter(leids, [nl + cs - 1], val, mask=mk)
            return nl + jnp.sum(mi)

        return lax.fori_loop(0, EB // 16, inner, nloc)

    nloc = lax.fori_loop(0, NB, p1, jnp.int32(0))
    ntk = (nloc + 15) >> 4

    def zh(k, _):
        for q in range(16):
            h[pl.ds(k * 256 + q * 16, 16)] = zero16i
        return 0

    lax.fori_loop(0, NPT // 16, zh, 0)

    def pA(t, _):
        lv = leids[pl.ds(t * 16, 16)]
        msk = (t * 16 + it16) < nloc
        v = lv >> 16
        addr = it16 * NPT + v
        c = plsc.load_gather(h, [addr], mask=msk)
        plsc.store_scatter(h, [addr], c + 1, mask=msk)
        return 0

    lax.fori_loop(0, ntk, pA, 0)

    def red(kb, _):
        def rl(L, acc):
            return acc + h[pl.ds(L * NPT + kb * 16, 16)]

        cnts[pl.ds(kb * 16, 16)] = lax.fori_loop(0, 16, rl, zero16i)
        return 0

    lax.fori_loop(0, NPT // 16, red, 0)

    carry = jnp.int32(0)
    for kb in range(NPT // 16):
        v = cnts[pl.ds(kb * 16, 16)]
        a = (v + 3) & ~jnp.int32(3)
        cs = plsc.cumsum(a)
        offs[pl.ds(kb * 16, 16)] = cs - a + carry
        carry = carry + jnp.max(cs)

    def rn(kb, _):
        def rl(L, run):
            tmp = h[pl.ds(L * NPT + kb * 16, 16)]
            h[pl.ds(L * NPT + kb * 16, 16)] = run
            return run + tmp

        lax.fori_loop(0, 16, rl, offs[pl.ds(kb * 16, 16)])
        return 0

    lax.fori_loop(0, NPT // 16, rn, 0)

    def pB(t, _):
        lv = leids[pl.ds(t * 16, 16)]
        msk = (t * 16 + it16) < nloc
        ev = lv & 0xFFFF
        v = lv >> 16
        addr = it16 * NPT + v
        p = plsc.load_gather(h, [addr], mask=msk)
        plsc.store_scatter(h, [addr], p + 1, mask=msk)
        plsc.store_scatter(posv, [ev], p, mask=msk)
        return 0

    lax.fori_loop(0, ntk, pB, 0)

    def p5(b, _):
        for c in range(4):
            pltpu.sync_copy(dh.at[pl.ds(c * E + b * EB, EB)],
                            dmcols.at[pl.ds(c * EB, EB)])
            pltpu.sync_copy(ms.at[pl.ds(c * E + b * EB, EB)],
                            dmcols.at[pl.ds((4 + c) * EB, EB)])

        def inner(c, _):
            pk = packed[pl.ds(b * EB + c * 16, 16)]
            s = pk & 0xFFFF
            dv = s + (pk >> 16)
            mk = (dv >= lo) & (dv < hi)
            p = posv[pl.ds(b * EB + c * 16, 16)]
            d0 = dmcols[pl.ds(c * 16, 16)]
            d1 = dmcols[pl.ds(EB + c * 16, 16)]
            d2 = dmcols[pl.ds(2 * EB + c * 16, 16)]
            d3 = dmcols[pl.ds(3 * EB + c * 16, 16)]
            m0 = dmcols[pl.ds(4 * EB + c * 16, 16)]
            m1 = dmcols[pl.ds(5 * EB + c * 16, 16)]
            m2 = dmcols[pl.ds(6 * EB + c * 16, 16)]
            m3 = dmcols[pl.ds(7 * EB + c * 16, 16)]
            s0 = 2 * p
            s1 = s0 + 1
            gi = 4 * s
            plsc.store_scatter(cidx, [s0], gi, mask=mk)
            plsc.store_scatter(cidx, [s1], gi + 2, mask=mk)
            plsc.store_scatter(cwr, [s0], jnp.where(m0 > 0.5, d0, negv),
                               mask=mk)
            plsc.store_scatter(cwr, [s1], jnp.where(m2 > 0.5, d2, negv),
                               mask=mk)
            plsc.store_scatter(cwf, [s0], jnp.where(m1 > 0.5, d1, negv),
                               mask=mk)
            plsc.store_scatter(cwf, [s1], jnp.where(m3 > 0.5, d3, negv),
                               mask=mk)
            return 0

        return lax.fori_loop(0, EB // 16, inner, 0)

    lax.fori_loop(0, NB, p5, 0)

    ones16 = jnp.ones((16,), jnp.float32)
    pltpu.sync_copy(inarr.at[pl.ds(2 * lo, 2 * NPT)],
                    atf.at[pl.ds(2 * lo, 2 * NPT)])
    for k in range(2 * NPT // 16):
        jl = 2 * lo + k * 16 + it16
        plsc.store_scatter(mssl, [2 * jl], atf[pl.ds(2 * lo + k * 16, 16)])
        plsc.store_scatter(mssl, [2 * jl + 1], ones16)

    @pl.when(sid > 0)
    def _init_halo():
        pltpu.sync_copy(inarr.at[pl.ds(2 * lo - 128, 128)],
                        atf.at[pl.ds(2 * lo - 128, 128)])
        for k in range(128 // 16):
            jl = 2 * lo - 128 + k * 16 + it16
            plsc.store_scatter(mssl, [2 * jl],
                               atf[pl.ds(2 * lo - 128 + k * 16, 16)])
            plsc.store_scatter(mssl, [2 * jl + 1], ones16)

    lt2 = it16 < 2
    lt4 = it16 < 4

    def stage_body(stage, _):
        @pl.when(stage == sid)
        def _active():
            @pl.when(stage > 0)
            def _pull():
                pltpu.sync_copy(mssh.at[pl.ds(4 * lo - 256, 256)],
                                mssl.at[pl.ds(4 * lo - 256, 256)])

            def one_node(i, o8):
                iv = jnp.full((16,), i, jnp.int32)
                cnt = jnp.max(plsc.load_gather(cnts, [iv]))
                cnt2 = 2 * cnt
                valid = it16 < cnt2
                gi = cidx[pl.ds(o8 * 8, 16)]
                wr = cwr[pl.ds(o8 * 8, 16)]
                wf = cwf[pl.ds(o8 * 8, 16)]
                mu = plsc.load_gather(mssl, [gi], mask=valid)
                su = plsc.load_gather(mssl, [gi + 1], mask=valid)
                rmr = jnp.where(valid, mu + wr, negv)
                rmf = jnp.where(valid, mu + wf, negv)
                rsr = jnp.where(valid, su, zf16)
                rsf = rsr

                def chunk(cc, carr):
                    crmr, crsr, crmf, crsf = carr
                    cvalid = (cc * 16 + it16) < cnt2
                    cgi = cidx[pl.ds(o8 * 8 + cc * 16, 16)]
                    cwrv = cwr[pl.ds(o8 * 8 + cc * 16, 16)]
                    cwfv = cwf[pl.ds(o8 * 8 + cc * 16, 16)]
                    cmu = plsc.load_gather(mssl, [cgi], mask=cvalid)
                    csu = jnp.where(
                        cvalid,
                        plsc.load_gather(mssl, [cgi + 1], mask=cvalid),
                        zf16)
                    vr = jnp.where(cvalid, cmu + cwrv, negv)
                    vf = jnp.where(cvalid, cmu + cwfv, negv)
                    nmr = jnp.maximum(crmr, vr)
                    nsr = crsr * jnp.exp((crmr - nmr) * INV_TAU) + \
                        csu * jnp.exp((vr - nmr) * INV_TAU)
                    nmf = jnp.maximum(crmf, vf)
                    nsf = crsf * jnp.exp((crmf - nmf) * INV_TAU) + \
                        csu * jnp.exp((vf - nmf) * INV_TAU)
                    return (nmr, nsr, nmf, nsf)

                rmr, rsr, rmf, rsf = lax.fori_loop(
                    1, (cnt2 + 15) >> 4, chunk, (rmr, rsr, rmf, rsf))

                mr = jnp.max(rmr)
                mrv = jnp.full((16,), mr, jnp.float32)
                srv = jnp.full((16,), jnp.sum(
                    rsr * jnp.exp((rmr - mrv) * INV_TAU)), jnp.float32)
                mf = jnp.max(rmf)
                mfv = jnp.full((16,), mf, jnp.float32)
                sfv = jnp.full((16,), jnp.sum(
                    rsf * jnp.exp((rmf - mfv) * INV_TAU)), jnp.float32)
                sv = jnp.where(lt2, srv, sfv)
                bits = plsc.bitcast(sv, jnp.int32)
                kk = ((bits >> 23) & 0xFF) - 127
                tt = plsc.bitcast((bits & 0x7FFFFF) | jnp.int32(0x3F800000),
                                  jnp.float32)
                mm = jnp.where(lt2, mrv, mfv) + \
                    (TAU * LN2) * kk.astype(jnp.float32)
                vals = jnp.where((it16 & 1) == 0, mm, tt)
                ok = lt4 & jnp.where(lt2, mr > -1e20, mf > -1e20)
                plsc.store_scatter(mssl, [4 * (lo + i) + it16], vals,
                                   mask=ok)
                return o8 + ((cnt + 3) >> 2)

            def node_body(g, o8):
                for k in range(4):
                    o8 = one_node(4 * g + k, o8)
                return o8

            lax.fori_loop(0, NPT // 4, node_body, jnp.int32(0))
            pltpu.sync_copy(mssl.at[pl.ds(4 * lo + 4 * NPT - 256, 256)],
                            mssh.at[pl.ds(4 * lo + 4 * NPT - 256, 256)])

        plsc.subcore_barrier()
        return 0

    lax.fori_loop(0, NTILES, stage_body, 0)

    for c in range(2 * NPT // 16):
        jl = 2 * lo + c * 16 + it16
        mval = plsc.load_gather(mssl, [2 * jl])
        sval = plsc.load_gather(mssl, [2 * jl + 1])
        atf[pl.ds(2 * lo + c * 16, 16)] = mval + TAU * _log16(sval)
    pltpu.sync_copy(atf.at[pl.ds(2 * lo, 2 * NPT)],
                    atsh.at[pl.ds(2 * lo, 2 * NPT)])
    pltpu.sync_copy(atf.at[pl.ds(2 * lo, 2 * NPT)],
                    out_at.at[pl.ds(2 * lo, 2 * NPT)])
    plsc.subcore_barrier()

    pltpu.sync_copy(atsh, atf)
    pltpu.sync_copy(epids.at[pl.ds(sid * EPT, EPT)], epv)
    pltpu.sync_copy(rat.at[pl.ds(sid * 2 * EPT, 2 * EPT)], ratv)
    for c in range(2 * EPT // 16):
        jl = c * 16 + it16
        row = jl >> 1
        ch = jl & 1
        ep = plsc.load_gather(epv, [row])
        a = plsc.load_gather(atf, [2 * ep + ch])
        r = plsc.load_gather(ratv, [jl])
        plsc.store_scatter(atepv, [jl], a)
        plsc.store_scatter(slkv, [jl], r - a)
    pltpu.sync_copy(atepv, out_ep.at[pl.ds(sid * 2 * EPT, 2 * EPT)])
    pltpu.sync_copy(slkv, out_slack.at[pl.ds(sid * 2 * EPT, 2 * EPT)])


def kernel(d_hat, sta_mask, edge_src, edge_dst, topo_order, input_arrival,
           endpoint_ids, rat_true):
    del topo_order
    at_flat, ep_flat, slk_flat = _sta_sc(
        d_hat.T.reshape(-1), sta_mask.T.reshape(-1), edge_src, edge_dst,
        input_arrival.reshape(-1), endpoint_ids, rat_true.reshape(-1))
    return (at_flat.reshape(N, 2), ep_flat.reshape(M, 2),
            slk_flat.reshape(M, 2))
